# Initial kernel scaffold; baseline (speedup 1.0000x reference)
import jax
import jax.numpy as jnp
from jax.experimental import pallas as pl


def kernel(x, attn_scores, relative_bias):
    raise NotImplementedError("write your pallas kernel here")


# speedup vs baseline: 94.2343x; 94.2343x over previous
import functools

import jax
import jax.numpy as jnp
from jax.experimental import pallas as pl
from jax.experimental.pallas import tpu as pltpu


def _round_up(n: int, m: int) -> int:
    return (n + m - 1) // m * m


def _bias_add_kernel(win_ref, attn_ref, out_ref, *, bi: int, wwin: int):
    seq = attn_ref.shape[3]
    row = win_ref[0, :, :]
    tile = jnp.broadcast_to(row, (bi, wwin))
    t = pltpu.roll(tile, (-(bi - 1)) % wwin, 1, stride=1, stride_axis=0)
    out_ref[0, 0, :, :] = attn_ref[0, 0, :, :] + t[:, :seq]


@jax.jit
def kernel(x, attn_scores, relative_bias):
    _, heads, seq, _ = attn_scores.shape
    bi = 512
    n_i = seq // bi
    wwin = _round_up(seq + bi - 1, 128)

    rb_t = relative_bias.T
    pad_to = _round_up(seq - bi + wwin, 128)
    rb_t = jnp.pad(rb_t, ((0, 0), (0, pad_to - rb_t.shape[1])))
    wins = jnp.stack(
        [rb_t[:, seq - bi * (1 + idx): seq - bi * (1 + idx) + wwin]
         for idx in range(n_i)], axis=1)
    wins = wins.reshape(heads * n_i, 1, wwin)

    out = pl.pallas_call(
        functools.partial(_bias_add_kernel, bi=bi, wwin=wwin),
        grid=(heads, n_i),
        in_specs=[
            pl.BlockSpec((1, 1, wwin), lambda h, i: (h * n_i + i, 0, 0)),
            pl.BlockSpec((1, 1, bi, seq), lambda h, i: (0, h, i, 0)),
        ],
        out_specs=pl.BlockSpec((1, 1, bi, seq), lambda h, i: (0, h, i, 0)),
        out_shape=jax.ShapeDtypeStruct(attn_scores.shape, attn_scores.dtype),
    )(wins, attn_scores)
    return out
